# Initial kernel scaffold; baseline (speedup 1.0000x reference)
import functools

import jax
import jax.numpy as jnp
from jax import lax
from jax.experimental import pallas as pl
from jax.experimental.pallas import tpu as pltpu
from jax.experimental.pallas import tpu_sc as plsc

N = 1048576
LANES = 16
NW = 32

SHIFT = 150000
A_BASE = 250000
A_VECS = 25000
B_BASE = 650000
B_VECS = 9375
C_BASE = 0
C_VECS = 15625
D_BASE = 800000
D_VECS = 15536

L_RAMP = 400000
M_LIN = 150000

A_STRIDE = 782
B_STRIDE = 293
C_STRIDE = 489
D_STRIDE = 486

_INV_L = 1.0 / (L_RAMP - 1)
_INV_M = 1.0 / (M_LIN - 1)


def _worker_off(wid, stride, total_vecs, base):
    v0 = jnp.minimum(wid * stride, total_vecs - stride)
    return pl.multiple_of(base + v0 * LANES, LANES)


def _body(sig, out, buf_a, buf_b, buf_c, buf_d, buf_s):
    wid = lax.axis_index("s") * 2 + lax.axis_index("c")
    iota = lax.iota(jnp.int32, (LANES,), 0)
    fiota = iota.astype(jnp.float32)

    pltpu.sync_copy(sig.at[pl.ds(250000, LANES)], buf_s.at[pl.ds(0, LANES)])
    pltpu.sync_copy(sig.at[pl.ds(400000, LANES)], buf_s.at[pl.ds(16, LANES)])
    pltpu.sync_copy(sig.at[pl.ds(649984, LANES)], buf_s.at[pl.ds(32, LANES)])
    pltpu.sync_copy(sig.at[pl.ds(799984, LANES)], buf_s.at[pl.ds(48, LANES)])

    def lane(vec, k):
        return jnp.sum(jnp.where(iota == k, vec, jnp.float32(0.0)))

    s250 = lane(buf_s[pl.ds(0, LANES)], 0)
    s400 = lane(buf_s[pl.ds(16, LANES)], 0)
    s649 = lane(buf_s[pl.ds(32, LANES)], 15)
    s799 = lane(buf_s[pl.ds(48, LANES)], 15)

    a = s400 - s250
    b = s799 - s649
    step = (b - a) * jnp.float32(_INV_L)
    step2 = b * jnp.float32(_INV_M)

    c_off = _worker_off(wid, C_STRIDE, C_VECS, C_BASE)
    pltpu.sync_copy(sig.at[pl.ds(c_off, C_STRIDE * LANES)], buf_c)
    pltpu.sync_copy(buf_c, out.at[pl.ds(c_off, C_STRIDE * LANES)])

    d_off = _worker_off(wid, D_STRIDE, D_VECS, D_BASE)
    pltpu.sync_copy(sig.at[pl.ds(d_off, D_STRIDE * LANES)], buf_d)
    pltpu.sync_copy(buf_d, out.at[pl.ds(d_off, D_STRIDE * LANES)])

    a_off = _worker_off(wid, A_STRIDE, A_VECS, A_BASE)
    pltpu.sync_copy(sig.at[pl.ds(a_off + SHIFT, A_STRIDE * LANES)], buf_a)
    ramp0 = a + step * ((a_off - A_BASE).astype(jnp.float32) + fiota)
    dstep = step * jnp.float32(LANES)

    def a_body(v, r):
        sl = pl.ds(v * LANES, LANES)
        buf_a[sl] = buf_a[sl] - r
        return r + dstep

    lax.fori_loop(0, A_STRIDE, a_body, ramp0, unroll=4)
    pltpu.sync_copy(buf_a, out.at[pl.ds(a_off, A_STRIDE * LANES)])

    b_off = _worker_off(wid, B_STRIDE, B_VECS, B_BASE)
    lin0 = s649 + step2 * ((b_off - B_BASE).astype(jnp.float32) + fiota)
    dstep2 = step2 * jnp.float32(LANES)

    def b_body(v, r):
        buf_b[pl.ds(v * LANES, LANES)] = r
        return r + dstep2

    lax.fori_loop(0, B_STRIDE, b_body, lin0, unroll=4)
    pltpu.sync_copy(buf_b, out.at[pl.ds(b_off, B_STRIDE * LANES)])


_delay_part = functools.partial(
    pl.kernel,
    out_type=jax.ShapeDtypeStruct((N,), jnp.float32),
    mesh=plsc.VectorSubcoreMesh(core_axis_name="c", subcore_axis_name="s"),
    scratch_types=[
        pltpu.VMEM((A_STRIDE * LANES,), jnp.float32),
        pltpu.VMEM((B_STRIDE * LANES,), jnp.float32),
        pltpu.VMEM((C_STRIDE * LANES,), jnp.float32),
        pltpu.VMEM((D_STRIDE * LANES,), jnp.float32),
        pltpu.VMEM((4 * LANES,), jnp.float32),
    ],
)(_body)


@jax.jit
def kernel(signal):
    return _delay_part(signal)


# speedup vs baseline: 134.8806x; 134.8806x over previous
import functools

import jax
import jax.numpy as jnp
from jax import lax
from jax.experimental import pallas as pl
from jax.experimental.pallas import tpu as pltpu
from jax.experimental.pallas import tpu_sc as plsc

N = 1048576
LANES = 16
NW = 32

SHIFT = 150000
A_BASE = 250000
A_VECS = 25000
B_BASE = 650000
B_VECS = 9375
C_BASE = 0
C_VECS = 15625
D_BASE = 800000
D_VECS = 15536

L_RAMP = 400000
M_LIN = 150000

A_STRIDE = 782
B_STRIDE = 293
C_STRIDE = 489
D_STRIDE = 486

_INV_L = 1.0 / (L_RAMP - 1)
_INV_M = 1.0 / (M_LIN - 1)


def _worker_off(wid, stride, total_vecs, base):
    v0 = jnp.minimum(wid * stride, total_vecs - stride)
    return pl.multiple_of(base + v0 * LANES, LANES)


def _body(sig, out, buf_a, buf_b, buf_c, buf_d, buf_s, buf_i):
    wid = lax.axis_index("s") * 2 + lax.axis_index("c")
    iota = lax.iota(jnp.int32, LANES)
    fiota = iota.astype(jnp.float32)

    def splat(pos):
        buf_i[...] = jnp.full((LANES,), pos, jnp.int32)
        pltpu.sync_copy(sig.at[buf_i], buf_s)
        return buf_s[...]

    s250 = splat(250000)
    s400 = splat(400000)
    s649 = splat(649999)
    s799 = splat(799999)

    a = s400 - s250
    b = s799 - s649
    step = (b - a) * jnp.float32(_INV_L)
    step2 = b * jnp.float32(_INV_M)

    c_off = _worker_off(wid, C_STRIDE, C_VECS, C_BASE)
    pltpu.sync_copy(sig.at[pl.ds(c_off, C_STRIDE * LANES)], buf_c)
    pltpu.sync_copy(buf_c, out.at[pl.ds(c_off, C_STRIDE * LANES)])

    d_off = _worker_off(wid, D_STRIDE, D_VECS, D_BASE)
    pltpu.sync_copy(sig.at[pl.ds(d_off, D_STRIDE * LANES)], buf_d)
    pltpu.sync_copy(buf_d, out.at[pl.ds(d_off, D_STRIDE * LANES)])

    a_off = _worker_off(wid, A_STRIDE, A_VECS, A_BASE)
    pltpu.sync_copy(sig.at[pl.ds(a_off + SHIFT, A_STRIDE * LANES)], buf_a)
    ramp0 = a + step * ((a_off - A_BASE).astype(jnp.float32) + fiota)
    dstep = step * jnp.float32(LANES)

    def a_body(v, r):
        sl = pl.ds(v * LANES, LANES)
        buf_a[sl] = buf_a[sl] - r
        return r + dstep

    lax.fori_loop(0, A_STRIDE, a_body, ramp0, unroll=4)
    pltpu.sync_copy(buf_a, out.at[pl.ds(a_off, A_STRIDE * LANES)])

    b_off = _worker_off(wid, B_STRIDE, B_VECS, B_BASE)
    lin0 = s649 + step2 * ((b_off - B_BASE).astype(jnp.float32) + fiota)
    dstep2 = step2 * jnp.float32(LANES)

    def b_body(v, r):
        buf_b[pl.ds(v * LANES, LANES)] = r
        return r + dstep2

    lax.fori_loop(0, B_STRIDE, b_body, lin0, unroll=4)
    pltpu.sync_copy(buf_b, out.at[pl.ds(b_off, B_STRIDE * LANES)])


_delay_part = functools.partial(
    pl.kernel,
    out_type=jax.ShapeDtypeStruct((N,), jnp.float32),
    mesh=plsc.VectorSubcoreMesh(core_axis_name="c", subcore_axis_name="s"),
    scratch_types=[
        pltpu.VMEM((A_STRIDE * LANES,), jnp.float32),
        pltpu.VMEM((B_STRIDE * LANES,), jnp.float32),
        pltpu.VMEM((C_STRIDE * LANES,), jnp.float32),
        pltpu.VMEM((D_STRIDE * LANES,), jnp.float32),
        pltpu.VMEM((LANES,), jnp.float32),
        pltpu.VMEM((LANES,), jnp.int32),
    ],
)(_body)


@jax.jit
def kernel(signal):
    return _delay_part(signal)
